# Initial kernel scaffold; baseline (speedup 1.0000x reference)
import functools

import jax
import jax.numpy as jnp
from jax import lax
from jax.experimental import pallas as pl
from jax.experimental.pallas import tpu as pltpu
import jax.experimental.pallas.tpu_sc as plsc

B = 4
N = 100000
C = 64
NX = 432
NY = 496
CELLS = NY * NX

LANES = 16
NWORK = 32
RANGE = CELLS // 8
PAD = N
TBL = N + LANES
CH = 6912
NCH = CELLS // CH
PCH = 4000
NPCH = N // PCH
SENTINEL = jnp.int32(1 << 30)

_MESH = dict(core_axis_name="c", subcore_axis_name="s")


def _cells_body(xs_ref, ys_ref, cells_ref):
    x = xs_ref[...]
    y = ys_ref[...]
    xmin = jnp.min(x, axis=1, keepdims=True)
    ymin = jnp.min(y, axis=1, keepdims=True)

    def fdiv(d, hi):
        mod = lax.rem(d, jnp.float32(0.16))
        q = lax.round((d - mod) / jnp.float32(0.16))
        return jnp.clip(q, 0.0, hi).astype(jnp.int32)

    xi = fdiv(x - xmin, 431.0)
    yi = fdiv(y - ymin, 495.0)
    cell = yi * NX + xi

    pos = lax.broadcasted_iota(jnp.int32, cell.shape, 1) % LANES
    dup = jnp.zeros(cell.shape, jnp.bool_)
    for s in range(1, LANES):
        shifted = jnp.concatenate(
            [cell[:, s:], jnp.full((cell.shape[0], s), -1, jnp.int32)], axis=1
        )
        dup = dup | ((cell == shifted) & (pos + s < LANES))
    cells_ref[...] = jnp.where(dup, SENTINEL, cell)


def _compute_cells(xs, ys):
    return pl.pallas_call(
        _cells_body,
        out_shape=jax.ShapeDtypeStruct((B, N), jnp.int32),
    )(xs, ys)


@functools.partial(
    pl.kernel,
    out_type=jax.ShapeDtypeStruct((B, CELLS), jnp.int32),
    mesh=plsc.VectorSubcoreMesh(**_MESH),
    scratch_types=[
        pltpu.VMEM((RANGE,), jnp.int32),
        pltpu.VMEM((PCH,), jnp.int32),
    ],
)
def _winner_map(cells_hbm, winner_hbm, win_v, cbuf):
    cid = lax.axis_index("c")
    sid = lax.axis_index("s")
    w = sid * 2 + cid
    b = w // 8
    base = (w % 8) * RANGE

    def init_body(i, carry):
        win_v[pl.ds(i * LANES, LANES)] = jnp.full((LANES,), PAD, jnp.int32)
        return carry

    lax.fori_loop(0, RANGE // LANES, init_body, 0)

    def chunk_body(k, carry):
        pltpu.sync_copy(cells_hbm.at[b, pl.ds(k * PCH, PCH)], cbuf)

        def vec_body(v, carry2):
            cells16 = cbuf[pl.ds(v * LANES, LANES)]
            local = cells16 - base
            mask = (local >= 0) & (local < RANGE)
            ivec = lax.iota(jnp.int32, LANES) + (k * PCH + v * LANES)
            plsc.store_scatter(win_v, [local], ivec, mask=mask)
            return carry2

        lax.fori_loop(0, PCH // LANES, vec_body, 0)
        return carry

    lax.fori_loop(0, NPCH, chunk_body, 0)
    pltpu.sync_copy(win_v, winner_hbm.at[b, pl.ds(base, RANGE)])


@functools.partial(
    pl.kernel,
    out_type=jax.ShapeDtypeStruct((B, C, CELLS), jnp.float32),
    mesh=plsc.VectorSubcoreMesh(**_MESH),
    scratch_types=[
        pltpu.VMEM((TBL,), jnp.float32),
        pltpu.VMEM((CH,), jnp.int32),
        pltpu.VMEM((CH,), jnp.float32),
    ],
)
def _scatter_planes(pf_hbm, winner_hbm, out_hbm, tbl_v, wbuf, obuf):
    cid = lax.axis_index("c")
    sid = lax.axis_index("s")
    w = sid * 2 + cid

    def plane_body(j, carry):
        p = w * 8 + j
        b = p // C
        c = p % C
        pltpu.sync_copy(pf_hbm.at[b, c], tbl_v.at[pl.ds(0, N)])
        tbl_v[pl.ds(N, LANES)] = jnp.zeros((LANES,), jnp.float32)

        def chunk_body(k, carry2):
            pltpu.sync_copy(winner_hbm.at[b, pl.ds(k * CH, CH)], wbuf)

            def vec_body(v, carry3):
                idx = wbuf[pl.ds(v * LANES, LANES)]
                obuf[pl.ds(v * LANES, LANES)] = plsc.load_gather(tbl_v, [idx])
                return carry3

            lax.fori_loop(0, CH // LANES, vec_body, 0)
            pltpu.sync_copy(obuf, out_hbm.at[b, c, pl.ds(k * CH, CH)])
            return carry2

        lax.fori_loop(0, NCH, chunk_body, 0)
        return carry

    lax.fori_loop(0, C * B // NWORK, plane_body, 0)


def kernel(point_feature, voxel_coords, points):
    del voxel_coords
    xs = points[:, 1].reshape(B, N)
    ys = points[:, 2].reshape(B, N)
    cells = _compute_cells(xs, ys)
    winner = _winner_map(cells)
    out = _scatter_planes(point_feature, winner)
    return out.reshape(B, C, NY, NX)


# speedup vs baseline: 1.9287x; 1.9287x over previous
import functools

import jax
import jax.numpy as jnp
from jax import lax
from jax.experimental import pallas as pl
from jax.experimental.pallas import tpu as pltpu
import jax.experimental.pallas.tpu_sc as plsc

B = 4
N = 100000
C = 64
NX = 432
NY = 496
CELLS = NY * NX

LANES = 16
NWORK = 32
RANGE = CELLS // 8
PAD = N
TBL = N + LANES
CH = 6912
NCH = CELLS // CH
PCH = 4000
NPCH = N // PCH
SENTINEL = 1 << 30


G = N // LANES


def _cells_body(xs_ref, ys_ref, cells_ref):
    x = xs_ref[0]
    y = ys_ref[0]
    xmin = jnp.min(x)
    ymin = jnp.min(y)

    def fdiv(d, hi):
        mod = lax.rem(d, jnp.float32(0.16))
        q = lax.round((d - mod) / jnp.float32(0.16))
        return jnp.clip(q, 0.0, hi).astype(jnp.int32)

    xi = fdiv(x - xmin, 431.0)
    yi = fdiv(y - ymin, 495.0)
    cell = yi * NX + xi

    dup = jnp.zeros(cell.shape, jnp.bool_)
    for s in range(1, LANES):
        shifted = jnp.concatenate(
            [cell[s:, :], jnp.full((s, G), -1, jnp.int32)], axis=0
        )
        rows = lax.broadcasted_iota(jnp.int32, cell.shape, 0) < (LANES - s)
        dup = dup | ((cell == shifted) & rows)
    cells_ref[0] = jnp.where(dup, SENTINEL, cell)


def _compute_cells(xs, ys):
    return pl.pallas_call(
        _cells_body,
        grid=(B,),
        in_specs=[
            pl.BlockSpec((1, LANES, G), lambda b: (b, 0, 0)),
            pl.BlockSpec((1, LANES, G), lambda b: (b, 0, 0)),
        ],
        out_specs=pl.BlockSpec((1, LANES, G), lambda b: (b, 0, 0)),
        out_shape=jax.ShapeDtypeStruct((B, LANES, G), jnp.int32),
    )(xs, ys)


@functools.cache
def _sc_kernels():
    mesh = plsc.VectorSubcoreMesh(core_axis_name="c", subcore_axis_name="s")
    params = pltpu.CompilerParams(needs_layout_passes=False)
    winner_map = pl.kernel(
        _winner_map_body,
        out_type=jax.ShapeDtypeStruct((B * CELLS,), jnp.int32),
        mesh=mesh,
        compiler_params=params,
        scratch_types=[
            pltpu.VMEM((RANGE,), jnp.int32),
            pltpu.VMEM((PCH,), jnp.int32),
        ],
    )
    scatter_planes = pl.kernel(
        _scatter_planes_body,
        out_type=jax.ShapeDtypeStruct((B * C * CELLS,), jnp.float32),
        mesh=mesh,
        compiler_params=params,
        scratch_types=[
            pltpu.VMEM((TBL,), jnp.float32),
            pltpu.VMEM((CH,), jnp.int32),
            pltpu.VMEM((CH,), jnp.float32),
        ],
    )
    return winner_map, scatter_planes


def _winner_map_body(cells_hbm, winner_hbm, win_v, cbuf):
    cid = lax.axis_index("c")
    sid = lax.axis_index("s")
    w = sid * 2 + cid
    b = w // 8
    base = (w % 8) * RANGE

    def init_body(i, carry):
        win_v[pl.ds(i * LANES, LANES)] = jnp.full((LANES,), PAD, jnp.int32)
        return carry

    lax.fori_loop(0, RANGE // LANES, init_body, 0)

    def chunk_body(k, carry):
        pltpu.sync_copy(cells_hbm.at[pl.ds(b * N + k * PCH, PCH)], cbuf)

        def vec_body(v, carry2):
            cells16 = cbuf[pl.ds(v * LANES, LANES)]
            local = cells16 - base
            mask = (local >= 0) & (local < RANGE)
            ivec = lax.iota(jnp.int32, LANES) + (k * PCH + v * LANES)
            plsc.store_scatter(win_v, [local], ivec, mask=mask)
            return carry2

        lax.fori_loop(0, PCH // LANES, vec_body, 0)
        return carry

    lax.fori_loop(0, NPCH, chunk_body, 0)
    pltpu.sync_copy(win_v, winner_hbm.at[pl.ds(b * CELLS + base, RANGE)])


def _scatter_planes_body(pf_hbm, winner_hbm, out_hbm, tbl_v, wbuf, obuf):
    cid = lax.axis_index("c")
    sid = lax.axis_index("s")
    w = sid * 2 + cid

    def plane_body(j, carry):
        p = w * 8 + j
        b = p // C
        pltpu.sync_copy(pf_hbm.at[pl.ds(p * N, N)], tbl_v.at[pl.ds(0, N)])
        tbl_v[pl.ds(N, LANES)] = jnp.zeros((LANES,), jnp.float32)

        def chunk_body(k, carry2):
            pltpu.sync_copy(winner_hbm.at[pl.ds(b * CELLS + k * CH, CH)], wbuf)

            def vec_body(v, carry3):
                idx = wbuf[pl.ds(v * LANES, LANES)]
                obuf[pl.ds(v * LANES, LANES)] = plsc.load_gather(tbl_v, [idx])
                return carry3

            lax.fori_loop(0, CH // LANES, vec_body, 0)
            pltpu.sync_copy(obuf, out_hbm.at[pl.ds(p * CELLS + k * CH, CH)])
            return carry2

        lax.fori_loop(0, NCH, chunk_body, 0)
        return carry

    lax.fori_loop(0, C * B // NWORK, plane_body, 0)


def kernel(point_feature, voxel_coords, points):
    del voxel_coords
    xs = points[:, 1].reshape(B, G, LANES).transpose(0, 2, 1)
    ys = points[:, 2].reshape(B, G, LANES).transpose(0, 2, 1)
    winner_map, scatter_planes = _sc_kernels()
    cells = _compute_cells(xs, ys).transpose(0, 2, 1).reshape(B * N)
    winner = winner_map(cells)
    out = scatter_planes(point_feature.reshape(B * C * N), winner)
    return out.reshape(B, C, NY, NX)


# speedup vs baseline: 3.1172x; 1.6162x over previous
import functools

import jax
import jax.numpy as jnp
from jax import lax
from jax.experimental import pallas as pl
from jax.experimental.pallas import tpu as pltpu
import jax.experimental.pallas.tpu_sc as plsc

B = 4
N = 100000
C = 64
NX = 432
NY = 496
CELLS = NY * NX

LANES = 16
NWORK = 32
RANGE = CELLS // 8
PAD = N
TBL = N + LANES
ROWS = 8
CH = ROWS * NX
NCH = NY // ROWS
VPR = NX // LANES
PCH = 4000
NPCH = N // PCH
SENTINEL = 1 << 30


G = N // LANES


def _cells_body(xs_ref, ys_ref, cells_ref):
    x = xs_ref[0]
    y = ys_ref[0]
    xmin = jnp.min(x)
    ymin = jnp.min(y)

    def fdiv(d, hi):
        mod = lax.rem(d, jnp.float32(0.16))
        q = lax.round((d - mod) / jnp.float32(0.16))
        return jnp.clip(q, 0.0, hi).astype(jnp.int32)

    xi = fdiv(x - xmin, 431.0)
    yi = fdiv(y - ymin, 495.0)
    cell = yi * NX + xi

    dup = jnp.zeros(cell.shape, jnp.bool_)
    for s in range(1, LANES):
        shifted = jnp.concatenate(
            [cell[s:, :], jnp.full((s, G), -1, jnp.int32)], axis=0
        )
        rows = lax.broadcasted_iota(jnp.int32, cell.shape, 0) < (LANES - s)
        dup = dup | ((cell == shifted) & rows)
    cells_ref[0] = jnp.where(dup, SENTINEL, cell)


def _compute_cells(xs, ys):
    return pl.pallas_call(
        _cells_body,
        grid=(B,),
        in_specs=[
            pl.BlockSpec((1, LANES, G), lambda b: (b, 0, 0)),
            pl.BlockSpec((1, LANES, G), lambda b: (b, 0, 0)),
        ],
        out_specs=pl.BlockSpec((1, LANES, G), lambda b: (b, 0, 0)),
        out_shape=jax.ShapeDtypeStruct((B, LANES, G), jnp.int32),
    )(xs, ys)


@functools.cache
def _sc_kernels():
    mesh = plsc.VectorSubcoreMesh(core_axis_name="c", subcore_axis_name="s")
    params = pltpu.CompilerParams(needs_layout_passes=False)
    winner_map = pl.kernel(
        _winner_map_body,
        out_type=jax.ShapeDtypeStruct((B * CELLS,), jnp.int32),
        mesh=mesh,
        compiler_params=params,
        scratch_types=[
            pltpu.VMEM((RANGE,), jnp.int32),
            pltpu.VMEM((PCH,), jnp.int32),
        ],
    )
    scatter_planes = pl.kernel(
        _scatter_planes_body,
        out_type=jax.ShapeDtypeStruct((B, C, NY, NX), jnp.float32),
        mesh=mesh,
        compiler_params=params,
        scratch_types=[
            pltpu.VMEM((TBL,), jnp.float32),
            pltpu.VMEM((2, CH), jnp.int32),
            pltpu.VMEM((2, ROWS, NX), jnp.float32),
            pltpu.SemaphoreType.DMA,
            pltpu.SemaphoreType.DMA,
            pltpu.SemaphoreType.DMA,
            pltpu.SemaphoreType.DMA,
        ],
    )
    return winner_map, scatter_planes


def _winner_map_body(cells_hbm, winner_hbm, win_v, cbuf):
    cid = lax.axis_index("c")
    sid = lax.axis_index("s")
    w = sid * 2 + cid
    b = w // 8
    base = (w % 8) * RANGE

    def init_body(i, carry):
        win_v[pl.ds(i * LANES, LANES)] = jnp.full((LANES,), PAD, jnp.int32)
        return carry

    lax.fori_loop(0, RANGE // LANES, init_body, 0)

    def chunk_body(k, carry):
        pltpu.sync_copy(cells_hbm.at[pl.ds(b * N + k * PCH, PCH)], cbuf)

        def vec_body(v, carry2):
            cells16 = cbuf[pl.ds(v * LANES, LANES)]
            local = cells16 - base
            mask = (local >= 0) & (local < RANGE)
            ivec = lax.iota(jnp.int32, LANES) + (k * PCH + v * LANES)
            plsc.store_scatter(win_v, [local], ivec, mask=mask)
            return carry2

        lax.fori_loop(0, PCH // LANES, vec_body, 0)
        return carry

    lax.fori_loop(0, NPCH, chunk_body, 0)
    pltpu.sync_copy(win_v, winner_hbm.at[pl.ds(b * CELLS + base, RANGE)])


def _scatter_planes_body(
    pf_hbm, winner_hbm, out_hbm, tbl_v, wbuf, obuf, ws0, ws1, os0, os1
):
    cid = lax.axis_index("c")
    sid = lax.axis_index("s")
    w = sid * 2 + cid
    wsems = (ws0, ws1)
    osems = (os0, os1)

    def start_w(b, k, slot, sem):
        pltpu.async_copy(
            winner_hbm.at[pl.ds(b * CELLS + k * CH, CH)], wbuf.at[slot], sem
        )

    def wait_w(slot, sem):
        pltpu.make_async_copy(
            winner_hbm.at[pl.ds(0, CH)], wbuf.at[slot], sem
        ).wait()

    def start_o(b, c, k, slot, sem):
        pltpu.async_copy(
            obuf.at[slot], out_hbm.at[b, c, pl.ds(k * ROWS, ROWS), :], sem
        )

    def wait_o(slot, sem):
        pltpu.make_async_copy(
            obuf.at[slot], out_hbm.at[0, 0, pl.ds(0, ROWS), :], sem
        ).wait()

    def gather_chunk(slot):
        def vec_body(v, carry):
            idx = wbuf[slot, pl.ds(v * LANES, LANES)]
            r = v // VPR
            q = v % VPR
            obuf[slot, r, pl.ds(q * LANES, LANES)] = plsc.load_gather(
                tbl_v, [idx]
            )
            return carry

        lax.fori_loop(0, CH // LANES, vec_body, 0)

    def plane_body(j, carry):
        p = w * 8 + j
        b = p // C
        c = p % C
        pltpu.sync_copy(pf_hbm.at[pl.ds(p * N, N)], tbl_v.at[pl.ds(0, N)])
        tbl_v[pl.ds(N, LANES)] = jnp.zeros((LANES,), jnp.float32)

        start_w(b, 0, 0, wsems[0])

        def pair_body(m, carry2):
            for slot in (0, 1):
                k = m * 2 + slot
                wait_w(slot, wsems[slot])

                @pl.when(k + 1 < NCH)
                def _():
                    start_w(b, k + 1, 1 - slot, wsems[1 - slot])

                @pl.when(m > 0)
                def _():
                    wait_o(slot, osems[slot])

                gather_chunk(slot)
                start_o(b, c, k, slot, osems[slot])
            return carry2

        lax.fori_loop(0, NCH // 2, pair_body, 0)
        wait_o(0, osems[0])
        wait_o(1, osems[1])
        return carry

    lax.fori_loop(0, C * B // NWORK, plane_body, 0)


def kernel(point_feature, voxel_coords, points):
    del voxel_coords
    xs = points[:, 1].reshape(B, G, LANES).transpose(0, 2, 1)
    ys = points[:, 2].reshape(B, G, LANES).transpose(0, 2, 1)
    winner_map, scatter_planes = _sc_kernels()
    cells = _compute_cells(xs, ys).transpose(0, 2, 1).reshape(B * N)
    winner = winner_map(cells)
    return scatter_planes(point_feature.reshape(B * C * N), winner)
